# baseline (speedup 1.0000x reference)
import functools

import jax
import jax.numpy as jnp
from jax import lax
from jax.experimental import pallas as pl
from jax.experimental.pallas import tpu as pltpu
from jax.experimental.pallas import tpu_sc as plsc

NUM_ENTITIES = 1000000
D = 64
DP = 128
B = 4096
F = 100
FP = 104

NC, NS = 2, 16
NW = NC * NS
BW = B // NW


@functools.partial(
    pl.kernel,
    out_type=(
        jax.ShapeDtypeStruct((F, D, B), jnp.float32),
        jax.ShapeDtypeStruct((F, D, B), jnp.float32),
    ),
    mesh=plsc.VectorSubcoreMesh(core_axis_name="c", subcore_axis_name="s"),
    compiler_params=pltpu.CompilerParams(
        use_tc_tiling_on_sc=True, needs_layout_passes=False),
    scratch_types=[
        pltpu.VMEM((FP, BW), jnp.int32),
        pltpu.VMEM((FP, BW), jnp.int32),
        pltpu.VMEM((BW, DP), jnp.float32),
        pltpu.VMEM((BW, DP), jnp.float32),
        pltpu.VMEM((BW, DP), jnp.float32),
        pltpu.VMEM((BW, DP), jnp.float32),
        pltpu.VMEM((D, BW), jnp.float32),
        pltpu.VMEM((D, BW), jnp.float32),
        pltpu.VMEM((D, BW), jnp.float32),
        pltpu.VMEM((D, BW), jnp.float32),
        pltpu.SemaphoreType.DMA,
        pltpu.SemaphoreType.DMA,
        pltpu.SemaphoreType.DMA,
        pltpu.SemaphoreType.DMA,
    ],
)
def _bracket_gather(jrow_hbm, half_hbm, bra_hbm, ket_hbm, bra_out, ket_out,
                    jrow_v, half_v, ga0, ga1, gk0, gk1, oa0, oa1, ok0, ok1,
                    gsem0, gsem1, ssem0, ssem1):
    wid = lax.axis_index("s") * NC + lax.axis_index("c")
    b0 = wid * BW
    gbuf = ((ga0, gk0), (ga1, gk1))
    obuf = ((oa0, ok0), (oa1, ok1))
    gsem = (gsem0, gsem1)
    ssem = (ssem0, ssem1)

    pltpu.sync_copy(jrow_hbm.at[:, pl.ds(b0, BW)], jrow_v)
    pltpu.sync_copy(half_hbm.at[:, pl.ds(b0, BW)], half_v)

    lanes = lax.iota(jnp.int32, 16)

    def fire_gathers(f, s):
        pltpu.async_copy(bra_hbm.at[jrow_v.at[f]], gbuf[s][0], gsem[s])
        pltpu.async_copy(ket_hbm.at[jrow_v.at[f]], gbuf[s][1], gsem[s])

    def wait_gathers(s):
        pltpu.make_async_copy(
            bra_hbm.at[jrow_v.at[0]], gbuf[s][0], gsem[s]).wait()
        pltpu.make_async_copy(
            ket_hbm.at[jrow_v.at[0]], gbuf[s][1], gsem[s]).wait()

    def fire_stores(f, s):
        pltpu.async_copy(obuf[s][0], bra_out.at[f, :, pl.ds(b0, BW)], ssem[s])
        pltpu.async_copy(obuf[s][1], ket_out.at[f, :, pl.ds(b0, BW)], ssem[s])

    def wait_stores(s):
        pltpu.make_async_copy(
            obuf[s][0], bra_out.at[0, :, pl.ds(0, BW)], ssem[s]).wait()
        pltpu.make_async_copy(
            obuf[s][1], ket_out.at[0, :, pl.ds(0, BW)], ssem[s]).wait()

    def transpose_block(f, s):
        offs = [half_v[f, pl.ds(16 * k, 16)] for k in range(8)]
        rows = [lanes + (16 * k) for k in range(8)]
        gb, gk_ = gbuf[s]
        ob, ok_ = obuf[s]

        def body(d, carry):
            for k in range(8):
                col = offs[k] + d
                vb = plsc.load_gather(gb, [rows[k], col])
                vk = plsc.load_gather(gk_, [rows[k], col])
                ob[d, pl.ds(16 * k, 16)] = vb
                ok_[d, pl.ds(16 * k, 16)] = vk
            return carry

        lax.fori_loop(0, D, body, 0)

    fire_gathers(0, 0)

    def pair(p, carry):
        for parity in range(2):
            f = p * 2 + parity
            other = 1 - parity
            wait_gathers(parity)
            @pl.when(f + 1 < F)
            def _():
                fire_gathers(f + 1, other)
            @pl.when(f >= 2)
            def _():
                wait_stores(parity)
            transpose_block(f, parity)
            fire_stores(f, parity)
        return carry

    lax.fori_loop(0, F // 2, pair, 0)
    wait_stores(0)
    wait_stores(1)


def kernel(index, bra_weight, ket_weight):
    idx = index.astype(jnp.int32)
    jrow = jnp.pad((idx >> 1).T, ((0, FP - F), (0, 0)))
    half = jnp.pad(((idx & 1) << 6).T, ((0, FP - F), (0, 0)))
    bra_c = bra_weight.reshape(NUM_ENTITIES // 2, DP)
    ket_c = ket_weight.reshape(NUM_ENTITIES // 2, DP)
    bra3, ket3 = _bracket_gather(jrow, half, bra_c, ket_c)
    return (jnp.transpose(bra3, (2, 0, 1)), jnp.transpose(ket3, (2, 0, 1)))
